# Initial kernel scaffold; baseline (speedup 1.0000x reference)
import jax
import jax.numpy as jnp
from jax.experimental import pallas as pl


def kernel(x, adj_indices, adj_values, W, b, prelu_alpha):
    raise NotImplementedError("write your pallas kernel here")


# speedup vs baseline: 3.4112x; 3.4112x over previous
import functools
import jax
import jax.numpy as jnp
from jax import lax
from jax.experimental import pallas as pl
from jax.experimental.pallas import tpu as pltpu
from jax.experimental.pallas import tpu_sc as plsc

NC = 2
NS = 16
NW = NC * NS
CHUNK = 128
LANE = 16

_GDN = lax.GatherDimensionNumbers(
    offset_dims=(), collapsed_slice_dims=(0,), start_index_map=(0,))


def _proj_body(x_ref, w_ref, b_ref, o_ref):
    o_ref[...] = lax.dot_general(
        x_ref[...], w_ref[...], (((1,), (1,)), ((), ())),
        preferred_element_type=jnp.float32) + b_ref[...]


def _combine_body(p_ref, a_ref, o_ref):
    s = p_ref[0] + p_ref[1]
    alpha = a_ref[0, 0]
    o_ref[...] = jnp.maximum(s, 0.0) + alpha * jnp.minimum(s, 0.0)


def _spmm_body(d, cpt, ncp, rows_per_tile,
               proj_hbm, packed_hbm, val_hbm, zeros_hbm, out_hbm,
               acc, pbuf, valbuf, rowsbuf, gsem0, gsem1):
    c = lax.axis_index("c")
    s = lax.axis_index("s")
    w = c * NS + s
    base = w * cpt

    pltpu.sync_copy(zeros_hbm, acc.at[pl.ds(s * rows_per_tile, rows_per_tile)])
    plsc.subcore_barrier()

    def scale(b):
        def group(g, cc):
            vv = valbuf[b, pl.ds(g * LANE, LANE)]
            for l in range(LANE):
                ve = lax.gather(
                    vv, jnp.full((LANE, 1), l, jnp.int32), _GDN, (1,),
                    mode=lax.GatherScatterMode.PROMISE_IN_BOUNDS)
                e = g * LANE + l
                for q in range(d // LANE):
                    sl = rowsbuf[b, e, pl.ds(q * LANE, LANE)]
                    rowsbuf[b, e, pl.ds(q * LANE, LANE)] = sl * ve
            return cc
        lax.fori_loop(0, CHUNK // LANE, group, 0)

    def consume(b, gsem):
        pltpu.make_async_copy(
            proj_hbm.at[pbuf.at[b, 0]], rowsbuf.at[b], gsem).wait()
        scale(b)
        pltpu.sync_copy(rowsbuf.at[b], acc.at[pbuf.at[b, 1]], add=True)

    def prep(k, b, gsem):
        pltpu.sync_copy(packed_hbm.at[k], pbuf.at[b])
        pltpu.sync_copy(val_hbm.at[k], valbuf.at[b])
        pltpu.async_copy(proj_hbm.at[pbuf.at[b, 0]], rowsbuf.at[b], gsem)

    prep(base, 0, gsem0)

    def pair(i0, carry):
        k0 = base + i0 * 2
        prep(k0 + 1, 1, gsem1)
        consume(0, gsem0)
        prep(lax.rem(k0 + 2, ncp), 0, gsem0)
        consume(1, gsem1)
        return carry

    lax.fori_loop(0, cpt // 2, pair, 0)
    pltpu.make_async_copy(
        proj_hbm.at[pbuf.at[0, 0]], rowsbuf.at[0], gsem0).wait()
    plsc.subcore_barrier()

    for t in range(rows_per_tile // CHUNK):
        r = s * rows_per_tile + t * CHUNK
        pltpu.sync_copy(acc.at[pl.ds(r, CHUNK)], rowsbuf.at[0])
        pltpu.sync_copy(rowsbuf.at[0], out_hbm.at[c, pl.ds(r, CHUNK)])


def kernel(x, adj_indices, adj_values, W, b, prelu_alpha):
    n, d_in = x.shape
    d_out = W.shape[0]
    e = adj_values.shape[0]

    blk = 2000
    nblk = n // blk
    proj = pl.pallas_call(
        _proj_body,
        grid=(nblk,),
        in_specs=[
            pl.BlockSpec((blk, d_in), lambda i: (i, 0)),
            pl.BlockSpec((d_out, d_in), lambda i: (0, 0)),
            pl.BlockSpec((1, d_out), lambda i: (0, 0)),
        ],
        out_specs=pl.BlockSpec((blk, d_out), lambda i: (i, 0)),
        out_shape=jax.ShapeDtypeStruct((n, d_out), jnp.float32),
    )(x, W, b.reshape(1, d_out))

    cpt = 2 * (-(-e // (NW * CHUNK * 2)))
    e_pad = NW * CHUNK * cpt
    ncp = e_pad // CHUNK
    pad = e_pad - e
    col = jnp.concatenate([adj_indices[1], jnp.zeros((pad,), jnp.int32)])
    dst = jnp.concatenate([adj_indices[0], jnp.zeros((pad,), jnp.int32)])
    val = jnp.concatenate([adj_values, jnp.zeros((pad,), jnp.float32)])
    packed = jnp.stack(
        [col.reshape(ncp, CHUNK), dst.reshape(ncp, CHUNK)], axis=1)
    val = val.reshape(ncp, CHUNK)
    n_pad = -(-n // (NS * CHUNK)) * NS * CHUNK
    rows_per_tile = n_pad // NS
    zeros = jnp.zeros((rows_per_tile, d_out), jnp.float32)

    mesh = plsc.VectorSubcoreMesh(
        core_axis_name="c", subcore_axis_name="s",
        num_cores=NC, num_subcores=NS)
    spmm = pl.kernel(
        functools.partial(_spmm_body, d_out, cpt, ncp, rows_per_tile),
        out_type=jax.ShapeDtypeStruct((NC, n_pad, d_out), jnp.float32),
        mesh=mesh,
        scratch_types=[
            pltpu.VMEM_SHARED((n_pad, d_out), jnp.float32),
            pltpu.VMEM((2, 2, CHUNK), jnp.int32),
            pltpu.VMEM((2, CHUNK), jnp.float32),
            pltpu.VMEM((2, CHUNK, d_out), jnp.float32),
            pltpu.SemaphoreType.DMA,
            pltpu.SemaphoreType.DMA,
        ],
    )
    partial = spmm(proj, packed, val, zeros)

    out = pl.pallas_call(
        _combine_body,
        grid=(nblk,),
        in_specs=[
            pl.BlockSpec((NC, blk, d_out), lambda i: (0, i, 0)),
            pl.BlockSpec((1, 1), lambda i: (0, 0)),
        ],
        out_specs=pl.BlockSpec((blk, d_out), lambda i: (i, 0)),
        out_shape=jax.ShapeDtypeStruct((n, d_out), jnp.float32),
    )(partial, prelu_alpha.reshape(1, 1))
    return out
